# baseline (speedup 1.0000x reference)
import jax
import jax.numpy as jnp
from jax.experimental import pallas as pl
from jax.experimental.pallas import tpu as pltpu

_BR = 1000
_N = 125
_BAND = 125000
_NBUF = 4
_LOOKAHEAD = 2


def _copy_body(src, dst, buf, in_sems, out_sems):
    def in_cps(i):
        b = i % _NBUF
        return [
            pltpu.make_async_copy(
                src.at[pl.ds(s * _BAND + i * _BR, _BR)], buf.at[b, s],
                in_sems.at[b])
            for s in range(8)
        ]

    def out_cp(i):
        b = i % _NBUF
        return pltpu.make_async_copy(
            buf.at[b], dst.at[:, pl.ds(i * _BR, _BR), :], out_sems.at[b])

    def start_in(i):
        for cp in in_cps(i):
            cp.start()

    def wait_in(i):
        for cp in in_cps(i):
            cp.wait()

    for i in range(_LOOKAHEAD):
        start_in(i)
    for i in range(_N):
        wait_in(i)
        out_cp(i).start()
        nxt = i + _LOOKAHEAD
        if nxt < _N:
            if nxt >= _NBUF:
                out_cp(nxt - _NBUF).wait()
            start_in(nxt)
    for i in range(max(0, _N - _NBUF), _N):
        out_cp(i).wait()


def kernel(embeddings):
    rows, dim = embeddings.shape
    out = pl.pallas_call(
        _copy_body,
        out_shape=jax.ShapeDtypeStruct((8, rows // 8, dim), embeddings.dtype),
        in_specs=[pl.BlockSpec(memory_space=pl.ANY)],
        out_specs=pl.BlockSpec(memory_space=pl.ANY),
        scratch_shapes=[
            pltpu.VMEM((_NBUF, 8, _BR, dim), embeddings.dtype),
            pltpu.SemaphoreType.DMA((_NBUF,)),
            pltpu.SemaphoreType.DMA((_NBUF,)),
        ],
    )(embeddings)
    return out.reshape(rows, dim)
